# baseline (speedup 1.0000x reference)
import jax
import jax.numpy as jnp
from jax.experimental import pallas as pl

BATCH = 16384
DIM = 2048
LANES = 128
NBLK = DIM // LANES
BLOCK_ROWS = 512


def _body(idx_ref, z_ref, o_ref):
    z = z_ref[...]
    idx = idx_ref[0, 0, :]
    for ob in range(NBLK):
        sub = idx[ob * LANES:(ob + 1) * LANES]
        sub2 = jnp.broadcast_to(sub[None, :], (BLOCK_ROWS, LANES))
        local = jnp.bitwise_and(sub2, LANES - 1)
        src_blk = jnp.right_shift(sub2, 7)
        acc = jnp.zeros((BLOCK_ROWS, LANES), z.dtype)
        for ib in range(NBLK):
            g = jnp.take_along_axis(
                z[:, ib * LANES:(ib + 1) * LANES], local, axis=1)
            acc = jnp.where(src_blk == ib, g, acc)
        o_ref[:, ob * LANES:(ob + 1) * LANES] = acc


def kernel(z, perm_indices):
    idx3 = perm_indices.reshape(1, 1, DIM)
    grid = (BATCH // BLOCK_ROWS,)
    return pl.pallas_call(
        _body,
        grid=grid,
        in_specs=[
            pl.BlockSpec((1, 1, DIM), lambda i: (0, 0, 0)),
            pl.BlockSpec((BLOCK_ROWS, DIM), lambda i: (i, 0)),
        ],
        out_specs=pl.BlockSpec((BLOCK_ROWS, DIM), lambda i: (i, 0)),
        out_shape=jax.ShapeDtypeStruct((BATCH, DIM), z.dtype),
    )(idx3, z)
